# Initial kernel scaffold; baseline (speedup 1.0000x reference)
import jax
import jax.numpy as jnp
from jax.experimental import pallas as pl


def kernel(x, loc, loc_orig, idx_agg, agg_weight, H, W, dw_weight, dw_bias):
    raise NotImplementedError("write your pallas kernel here")


# speedup vs baseline: 4.0135x; 4.0135x over previous
import functools

import jax
import jax.numpy as jnp
from jax import lax
from jax.experimental import pallas as pl
from jax.experimental.pallas import tpu as pltpu
from jax.experimental.pallas import tpu_sc as plsc

B, N, C, N0 = 8, 4096, 96, 16384
H, W = 128, 128
HW = H * W

NC, NS, L = 2, 16, 16
BPC = B // NC
PPT = N0 // NS
CHUNK = 128
NCHUNK = PPT // CHUNK
MROWS = HW // NS
TROWS = N // NS
CW = 16
CG = C // L

_MAGIC = 8388608.0


def _pix1d(v, d):
    v = jnp.minimum(jnp.maximum(v, -1.0), 1.0)
    t = 0.5 * (v + 1.0) * float(d) - 0.5
    r = (t + _MAGIC) - _MAGIC
    i = r.astype(jnp.int32)
    return jnp.minimum(jnp.maximum(i, 0), d - 1)


def _compute_pix(lxv, lyv, pixv, base_off):
    for g in range(CHUNK // L):
        s = pl.ds(g * L, L)
        xi = _pix1d(lxv[s], W)
        yi = _pix1d(lyv[s], H)
        pixv[s] = xi + yi * W + base_off


def _phase1_body(x_ref, lx_ref, ly_ref, tok_ref, zrow_ref, zcnt_ref, ones_ref,
                 msum_ref, cnt_ref,
                 map_sh, cnt_sh, lxv, lyv, tokv, pixv, rows, onesv, sem):
    cid = lax.axis_index("c")
    sid = lax.axis_index("s")
    pltpu.sync_copy(ones_ref, onesv)
    for bi in range(BPC):
        b = cid * BPC + bi
        pltpu.sync_copy(zrow_ref, map_sh.at[pl.ds(sid * MROWS, MROWS)])
        pltpu.sync_copy(zcnt_ref, cnt_sh.at[pl.ds(sid * MROWS, MROWS)])
        plsc.subcore_barrier()

        def chunk_body(ch, _, b=b, sid=sid):
            base = b * N0 + sid * PPT + ch * CHUNK
            pltpu.sync_copy(lx_ref.at[pl.ds(base, CHUNK)], lxv)
            pltpu.sync_copy(ly_ref.at[pl.ds(base, CHUNK)], lyv)
            pltpu.sync_copy(tok_ref.at[pl.ds(base, CHUNK)], tokv)
            _compute_pix(lxv, lyv, pixv, 0)
            for g in range(CHUNK // L):
                s = pl.ds(g * L, L)
                tokv[s] = tokv[s] + b * N
            pltpu.async_copy(x_ref.at[tokv], rows, sem).wait()
            pltpu.sync_copy(rows, map_sh.at[pixv], add=True)
            pltpu.sync_copy(onesv, cnt_sh.at[pixv], add=True)
            return 0

        lax.fori_loop(0, NCHUNK, chunk_body, 0)
        plsc.subcore_barrier()
        out_base = b * HW + sid * MROWS
        pltpu.sync_copy(map_sh.at[pl.ds(sid * MROWS, MROWS)],
                        msum_ref.at[pl.ds(out_base, MROWS)])
        pltpu.sync_copy(cnt_sh.at[pl.ds(sid * MROWS, MROWS)],
                        cnt_ref.at[pl.ds(out_base, MROWS)])


def _phase3_body(y_ref, lx_ref, ly_ref, tok_ref, w_ref, zrow_ref, zcnt_ref,
                 out_ref,
                 acc_sh, den_sh, lxv, lyv, tokv, pixv, wv, rows, wrows,
                 fbuf, dbuf, sem):
    cid = lax.axis_index("c")
    sid = lax.axis_index("s")
    for bi in range(BPC):
        b = cid * BPC + bi
        pltpu.sync_copy(zrow_ref.at[pl.ds(0, TROWS)],
                        acc_sh.at[pl.ds(sid * TROWS, TROWS)])
        pltpu.sync_copy(zcnt_ref.at[pl.ds(0, TROWS)],
                        den_sh.at[pl.ds(sid * TROWS, TROWS)])
        plsc.subcore_barrier()

        def chunk_body(ch, _, b=b, sid=sid):
            base = b * N0 + sid * PPT + ch * CHUNK
            pltpu.sync_copy(lx_ref.at[pl.ds(base, CHUNK)], lxv)
            pltpu.sync_copy(ly_ref.at[pl.ds(base, CHUNK)], lyv)
            pltpu.sync_copy(tok_ref.at[pl.ds(base, CHUNK)], tokv)
            pltpu.sync_copy(w_ref.at[pl.ds(base, CHUNK)], wv)
            _compute_pix(lxv, lyv, pixv, b * HW)
            pltpu.async_copy(y_ref.at[pixv], rows, sem).wait()

            def scale_body(j, _):
                wj = plsc.load_gather(wv, [jnp.full((L,), 0, jnp.int32) + j])
                wrows[j, :] = wj
                for c in range(CG):
                    s = pl.ds(c * L, L)
                    rows[j, s] = rows[j, s] * wj
                return 0

            lax.fori_loop(0, CHUNK, scale_body, 0)
            pltpu.sync_copy(rows, acc_sh.at[tokv], add=True)
            pltpu.sync_copy(wrows, den_sh.at[tokv], add=True)
            return 0

        lax.fori_loop(0, NCHUNK, chunk_body, 0)
        plsc.subcore_barrier()
        tb = sid * TROWS
        pltpu.sync_copy(acc_sh.at[pl.ds(tb, TROWS)], fbuf)
        pltpu.sync_copy(den_sh.at[pl.ds(tb, TROWS)], dbuf)

        def fin_body(j, _):
            r = 1.0 / (dbuf[j, :] + 1e-6)
            for c in range(CG):
                s = pl.ds(c * L, L)
                fbuf[j, s] = fbuf[j, s] * r
            return 0

        lax.fori_loop(0, TROWS, fin_body, 0)
        pltpu.sync_copy(fbuf, out_ref.at[pl.ds(b * N + tb, TROWS)])


def _conv_body(ms_ref, ct_ref, wk_ref, bias_ref, out_ref, pad_ref):
    xm = ms_ref[0] / (ct_ref[0][:, :, 0:1] + 1e-6)
    zr = jnp.zeros((1, W + 2, C), jnp.float32)
    zc = jnp.zeros((H, 1, C), jnp.float32)
    pad_ref[0:1, :, :] = zr
    pad_ref[H + 1:H + 2, :, :] = zr
    pad_ref[1:H + 1, 0:1, :] = zc
    pad_ref[1:H + 1, W + 1:W + 2, :] = zc
    pad_ref[1:H + 1, 1:W + 1, :] = xm
    acc = jnp.broadcast_to(bias_ref[0], (H, W, C))
    for dh in range(3):
        for dw in range(3):
            acc = acc + pad_ref[dh:dh + H, dw:dw + W, :] * wk_ref[dh, dw, :]
    out_ref[0] = acc


def _sc_mesh():
    return plsc.VectorSubcoreMesh(core_axis_name="c", subcore_axis_name="s",
                                  num_cores=NC, num_subcores=NS)


_f32 = jnp.float32


@jax.jit
def _run(x2, locx, locy, tokf, wf, wk, bias):
    zrow = jnp.zeros((MROWS, C), _f32)
    zcnt = jnp.zeros((MROWS, CW), _f32)
    ones_rows = jnp.ones((CHUNK, CW), _f32)

    phase1 = pl.kernel(
        _phase1_body,
        out_type=(jax.ShapeDtypeStruct((B * HW, C), _f32),
                  jax.ShapeDtypeStruct((B * HW, CW), _f32)),
        mesh=_sc_mesh(),
        compiler_params=pltpu.CompilerParams(use_tc_tiling_on_sc=False, needs_layout_passes=False),
        scratch_types=[
            pltpu.VMEM_SHARED((HW, C), _f32),
            pltpu.VMEM_SHARED((HW, CW), _f32),
            pltpu.VMEM((CHUNK,), _f32),
            pltpu.VMEM((CHUNK,), _f32),
            pltpu.VMEM((CHUNK,), jnp.int32),
            pltpu.VMEM((CHUNK,), jnp.int32),
            pltpu.VMEM((CHUNK, C), _f32),
            pltpu.VMEM((CHUNK, CW), _f32),
            pltpu.SemaphoreType.DMA,
        ],
    )
    msum, cnt = phase1(x2, locx, locy, tokf, zrow, zcnt, ones_rows)

    y = pl.pallas_call(
        _conv_body,
        grid=(B,),
        in_specs=[
            pl.BlockSpec((1, H, W, C), lambda b: (b, 0, 0, 0)),
            pl.BlockSpec((1, H, W, CW), lambda b: (b, 0, 0, 0)),
            pl.BlockSpec((3, 3, C), lambda b: (0, 0, 0)),
            pl.BlockSpec((1, C), lambda b: (0, 0)),
        ],
        out_specs=pl.BlockSpec((1, H, W, C), lambda b: (b, 0, 0, 0)),
        out_shape=jax.ShapeDtypeStruct((B, H, W, C), _f32),
        scratch_shapes=[pltpu.VMEM((H + 2, W + 2, C), _f32)],
    )(msum.reshape(B, H, W, C), cnt.reshape(B, H, W, CW), wk,
      bias.reshape(1, C))

    phase3 = pl.kernel(
        _phase3_body,
        out_type=jax.ShapeDtypeStruct((B * N, C), _f32),
        mesh=_sc_mesh(),
        compiler_params=pltpu.CompilerParams(use_tc_tiling_on_sc=False, needs_layout_passes=False),
        scratch_types=[
            pltpu.VMEM_SHARED((N, C), _f32),
            pltpu.VMEM_SHARED((N, CW), _f32),
            pltpu.VMEM((CHUNK,), _f32),
            pltpu.VMEM((CHUNK,), _f32),
            pltpu.VMEM((CHUNK,), jnp.int32),
            pltpu.VMEM((CHUNK,), jnp.int32),
            pltpu.VMEM((CHUNK,), _f32),
            pltpu.VMEM((CHUNK, C), _f32),
            pltpu.VMEM((CHUNK, CW), _f32),
            pltpu.VMEM((TROWS, C), _f32),
            pltpu.VMEM((TROWS, CW), _f32),
            pltpu.SemaphoreType.DMA,
        ],
    )
    out = phase3(y.reshape(B * HW, C), locx, locy, tokf, wf, zrow, zcnt)
    return out.reshape(B, N, C)


def kernel(x, loc, loc_orig, idx_agg, agg_weight, H_, W_, dw_weight, dw_bias):
    del loc
    x2 = x.reshape(B * N, C)
    locx = loc_orig[..., 0].reshape(B * N0)
    locy = loc_orig[..., 1].reshape(B * N0)
    tokf = idx_agg.astype(jnp.int32).reshape(B * N0)
    wf = agg_weight.astype(_f32).reshape(B * N0)
    wk = jnp.transpose(dw_weight[:, 0], (1, 2, 0))
    return _run(x2, locx, locy, tokf, wf, wk, dw_bias.astype(_f32))


# speedup vs baseline: 4.4524x; 1.1093x over previous
import jax
import jax.numpy as jnp
from jax import lax
from jax.experimental import pallas as pl
from jax.experimental.pallas import tpu as pltpu
from jax.experimental.pallas import tpu_sc as plsc

B, N, C, N0 = 8, 4096, 96, 16384
H, W = 128, 128
HW = H * W

NC, NS, L = 2, 16, 16
BPC = B // NC
PPT = N0 // NS
CH1 = 64
NCH1 = PPT // CH1
CH3 = 128
NCH3 = PPT // CH3
MROWS = HW // NS
TROWS = N // NS
CW = 16
CG = C // L

_f32 = jnp.float32


def _idx_body(lx_ref, ly_ref, tok_ref, pixloc_ref, tokabs_ref, pixabs_ref):
    def pix1d(v, d):
        v = jnp.minimum(jnp.maximum(v, -1.0), 1.0)
        t = 0.5 * (v + 1.0) * float(d) - 0.5
        i = jnp.round(t).astype(jnp.int32)
        return jnp.minimum(jnp.maximum(i, 0), d - 1)

    xi = pix1d(lx_ref[...], W)
    yi = pix1d(ly_ref[...], H)
    pix = xi + yi * W
    boff = lax.broadcasted_iota(jnp.int32, (B, N0), 0)
    pixloc_ref[...] = pix
    tokabs_ref[...] = tok_ref[...] + boff * N
    pixabs_ref[...] = pix + boff * HW


def _phase1_body(x_ref, pixloc_ref, tokabs_ref, zrow_ref, zcnt_ref, ones_ref,
                 msum_ref, cnt_ref,
                 map_sh, cnt_sh, pixst, tokst, rows2, onesv,
                 gsem, msem, csem):
    cid = lax.axis_index("c")
    sid = lax.axis_index("s")
    pltpu.sync_copy(ones_ref, onesv)
    for bi in range(BPC):
        b = cid * BPC + bi
        pltpu.sync_copy(zrow_ref, map_sh.at[pl.ds(sid * MROWS, MROWS)])
        pltpu.sync_copy(zcnt_ref, cnt_sh.at[pl.ds(sid * MROWS, MROWS)])
        row0 = (b * NS + sid) * NCH1
        pltpu.sync_copy(pixloc_ref.at[pl.ds(row0, NCH1)], pixst)
        pltpu.sync_copy(tokabs_ref.at[pl.ds(row0, NCH1)], tokst)
        plsc.subcore_barrier()

        for ch in range(NCH1):
            p = ch & 1
            pltpu.async_copy(x_ref.at[tokst.at[ch]], rows2.at[p], gsem).wait()
            pltpu.sync_copy(rows2.at[p], map_sh.at[pixst.at[ch]], add=True)
            pltpu.sync_copy(onesv, cnt_sh.at[pixst.at[ch]], add=True)
        plsc.subcore_barrier()
        out_base = b * HW + sid * MROWS
        pltpu.sync_copy(map_sh.at[pl.ds(sid * MROWS, MROWS)],
                        msum_ref.at[pl.ds(out_base, MROWS)])
        pltpu.sync_copy(cnt_sh.at[pl.ds(sid * MROWS, MROWS)],
                        cnt_ref.at[pl.ds(out_base, MROWS)])


def _phase3_body(y_ref, pixabs_ref, tokloc_ref, w_ref, zrow_ref, zcnt_ref,
                 out_ref,
                 acc_sh, den_sh, pixst, tokst, wb, rows2, wrows2, fbuf, dbuf,
                 gsem, msem, csem):
    cid = lax.axis_index("c")
    sid = lax.axis_index("s")
    for bi in range(BPC):
        b = cid * BPC + bi
        pltpu.sync_copy(zrow_ref.at[pl.ds(0, TROWS)],
                        acc_sh.at[pl.ds(sid * TROWS, TROWS)])
        pltpu.sync_copy(zcnt_ref.at[pl.ds(0, TROWS)],
                        den_sh.at[pl.ds(sid * TROWS, TROWS)])
        row0 = (b * NS + sid) * NCH3
        pltpu.sync_copy(pixabs_ref.at[pl.ds(row0, NCH3)], pixst)
        pltpu.sync_copy(tokloc_ref.at[pl.ds(row0, NCH3)], tokst)
        pltpu.sync_copy(w_ref.at[pl.ds(b * N0 + sid * PPT, PPT)], wb)
        plsc.subcore_barrier()

        for ch in range(NCH3):
            p = ch & 1
            pltpu.async_copy(y_ref.at[pixst.at[ch]], rows2.at[p], gsem).wait()

            def scale_body(j, _, p=p, ch=ch):
                wj = plsc.load_gather(
                    wb, [jnp.full((L,), ch * CH3, jnp.int32) + j])
                wrows2[p, j, :] = wj
                for c in range(CG):
                    s = pl.ds(c * L, L)
                    rows2[p, j, s] = rows2[p, j, s] * wj
                return 0

            lax.fori_loop(0, CH3, scale_body, 0)
            pltpu.sync_copy(rows2.at[p], acc_sh.at[tokst.at[ch]], add=True)
            pltpu.sync_copy(wrows2.at[p], den_sh.at[tokst.at[ch]], add=True)
        plsc.subcore_barrier()
        tb = sid * TROWS
        pltpu.sync_copy(acc_sh.at[pl.ds(tb, TROWS)], fbuf)
        pltpu.sync_copy(den_sh.at[pl.ds(tb, TROWS)], dbuf)

        def fin_body(j, _):
            r = 1.0 / (dbuf[j, :] + 1e-6)
            for c in range(CG):
                s = pl.ds(c * L, L)
                fbuf[j, s] = fbuf[j, s] * r
            return 0

        lax.fori_loop(0, TROWS, fin_body, 0)
        pltpu.sync_copy(fbuf, out_ref.at[pl.ds(b * N + tb, TROWS)])


def _conv_body(ms_ref, ct_ref, wk_ref, bias_ref, out_ref, pad_ref):
    xm = ms_ref[0] / (ct_ref[0][:, :, 0:1] + 1e-6)
    zr = jnp.zeros((1, W + 2, C), jnp.float32)
    zc = jnp.zeros((H, 1, C), jnp.float32)
    pad_ref[0:1, :, :] = zr
    pad_ref[H + 1:H + 2, :, :] = zr
    pad_ref[1:H + 1, 0:1, :] = zc
    pad_ref[1:H + 1, W + 1:W + 2, :] = zc
    pad_ref[1:H + 1, 1:W + 1, :] = xm
    acc = jnp.broadcast_to(bias_ref[0], (H, W, C))
    for dh in range(3):
        for dw in range(3):
            acc = acc + pad_ref[dh:dh + H, dw:dw + W, :] * wk_ref[dh, dw, :]
    out_ref[0] = acc


def _sc_mesh():
    return plsc.VectorSubcoreMesh(core_axis_name="c", subcore_axis_name="s",
                                  num_cores=NC, num_subcores=NS)


_SC_PARAMS = pltpu.CompilerParams(use_tc_tiling_on_sc=False,
                                  needs_layout_passes=False)


@jax.jit
def _run(x2, locx, locy, tokf, wf, wk, bias):
    zrow = jnp.zeros((MROWS, C), _f32)
    zcnt = jnp.zeros((MROWS, CW), _f32)
    ones_rows = jnp.ones((CH1, CW), _f32)

    pixloc, tokabs, pixabs = pl.pallas_call(
        _idx_body,
        out_shape=(jax.ShapeDtypeStruct((B, N0), jnp.int32),
                   jax.ShapeDtypeStruct((B, N0), jnp.int32),
                   jax.ShapeDtypeStruct((B, N0), jnp.int32)),
    )(locx.reshape(B, N0), locy.reshape(B, N0), tokf.reshape(B, N0))

    phase1 = pl.kernel(
        _phase1_body,
        out_type=(jax.ShapeDtypeStruct((B * HW, C), _f32),
                  jax.ShapeDtypeStruct((B * HW, CW), _f32)),
        mesh=_sc_mesh(),
        compiler_params=_SC_PARAMS,
        scratch_types=[
            pltpu.VMEM_SHARED((HW, C), _f32),
            pltpu.VMEM_SHARED((HW, CW), _f32),
            pltpu.VMEM((NCH1, CH1), jnp.int32),
            pltpu.VMEM((NCH1, CH1), jnp.int32),
            pltpu.VMEM((2, CH1, C), _f32),
            pltpu.VMEM((CH1, CW), _f32),
            pltpu.SemaphoreType.DMA,
            pltpu.SemaphoreType.DMA,
            pltpu.SemaphoreType.DMA,
        ],
    )
    msum, cnt = phase1(x2, pixloc.reshape(B * NS * NCH1, CH1),
                       tokabs.reshape(B * NS * NCH1, CH1),
                       zrow, zcnt, ones_rows)

    y = pl.pallas_call(
        _conv_body,
        grid=(B,),
        in_specs=[
            pl.BlockSpec((1, H, W, C), lambda b: (b, 0, 0, 0)),
            pl.BlockSpec((1, H, W, CW), lambda b: (b, 0, 0, 0)),
            pl.BlockSpec((3, 3, C), lambda b: (0, 0, 0)),
            pl.BlockSpec((1, C), lambda b: (0, 0)),
        ],
        out_specs=pl.BlockSpec((1, H, W, C), lambda b: (b, 0, 0, 0)),
        out_shape=jax.ShapeDtypeStruct((B, H, W, C), _f32),
        scratch_shapes=[pltpu.VMEM((H + 2, W + 2, C), _f32)],
    )(msum.reshape(B, H, W, C), cnt.reshape(B, H, W, CW), wk,
      bias.reshape(1, C))

    phase3 = pl.kernel(
        _phase3_body,
        out_type=jax.ShapeDtypeStruct((B * N, C), _f32),
        mesh=_sc_mesh(),
        compiler_params=_SC_PARAMS,
        scratch_types=[
            pltpu.VMEM_SHARED((N, C), _f32),
            pltpu.VMEM_SHARED((N, CW), _f32),
            pltpu.VMEM((NCH3, CH3), jnp.int32),
            pltpu.VMEM((NCH3, CH3), jnp.int32),
            pltpu.VMEM((PPT,), _f32),
            pltpu.VMEM((2, CH3, C), _f32),
            pltpu.VMEM((2, CH3, CW), _f32),
            pltpu.VMEM((TROWS, C), _f32),
            pltpu.VMEM((TROWS, CW), _f32),
            pltpu.SemaphoreType.DMA,
            pltpu.SemaphoreType.DMA,
            pltpu.SemaphoreType.DMA,
        ],
    )
    out = phase3(y.reshape(B * HW, C), pixabs.reshape(B * NS * NCH3, CH3),
                 tokf.reshape(B * NS * NCH3, CH3), wf, zrow, zcnt)
    return out.reshape(B, N, C)


def kernel(x, loc, loc_orig, idx_agg, agg_weight, H_, W_, dw_weight, dw_bias):
    del loc
    x2 = x.reshape(B * N, C)
    locx = loc_orig[..., 0].reshape(B * N0)
    locy = loc_orig[..., 1].reshape(B * N0)
    tokf = idx_agg.astype(jnp.int32).reshape(B * N0)
    wf = agg_weight.astype(_f32).reshape(B * N0)
    wk = jnp.transpose(dw_weight[:, 0], (1, 2, 0))
    return _run(x2, locx, locy, tokf, wf, wk, dw_bias.astype(_f32))


# speedup vs baseline: 4.8437x; 1.0879x over previous
import jax
import jax.numpy as jnp
from jax import lax
from jax.experimental import pallas as pl
from jax.experimental.pallas import tpu as pltpu
from jax.experimental.pallas import tpu_sc as plsc

B, N, C, N0 = 8, 4096, 96, 16384
H, W = 128, 128
HW = H * W

NC, NS, L = 2, 16, 16
BPC = B // NC
PPT = N0 // NS
CH1 = 64
NCH1 = PPT // CH1
CH3 = 128
NCH3 = PPT // CH3
MROWS = HW // NS
TROWS = N // NS
CW = 16
CG = C // L

_f32 = jnp.float32


def _idx_body(lx_ref, ly_ref, tok_ref, pixloc_ref, tokabs_ref, pixabs_ref):
    def pix1d(v, d):
        v = jnp.minimum(jnp.maximum(v, -1.0), 1.0)
        t = 0.5 * (v + 1.0) * float(d) - 0.5
        i = jnp.round(t).astype(jnp.int32)
        return jnp.minimum(jnp.maximum(i, 0), d - 1)

    xi = pix1d(lx_ref[...], W)
    yi = pix1d(ly_ref[...], H)
    pix = xi + yi * W
    boff = lax.broadcasted_iota(jnp.int32, (B, N0), 0)
    pixloc_ref[...] = pix
    tokabs_ref[...] = tok_ref[...] + boff * N
    pixabs_ref[...] = pix + boff * HW


def _phase1_body(x_ref, pixloc_ref, tokabs_ref, zrow_ref, zcnt_ref, ones_ref,
                 msum_ref, cnt_ref,
                 map_sh, cnt_sh, pixst, tokst, rows2, onesv,
                 gsem, msem, csem):
    cid = lax.axis_index("c")
    sid = lax.axis_index("s")
    pltpu.sync_copy(ones_ref, onesv)
    for bi in range(BPC):
        b = cid * BPC + bi
        pltpu.sync_copy(zrow_ref, map_sh.at[pl.ds(sid * MROWS, MROWS)])
        pltpu.sync_copy(zcnt_ref, cnt_sh.at[pl.ds(sid * MROWS, MROWS)])
        row0 = (b * NS + sid) * NCH1
        pltpu.sync_copy(pixloc_ref.at[pl.ds(row0, NCH1)], pixst)
        pltpu.sync_copy(tokabs_ref.at[pl.ds(row0, NCH1)], tokst)
        plsc.subcore_barrier()

        g = pltpu.async_copy(x_ref.at[tokst.at[0]], rows2.at[0], gsem)
        for ch in range(NCH1):
            p = ch & 1
            g.wait()
            if ch + 1 < NCH1:
                g = pltpu.async_copy(x_ref.at[tokst.at[ch + 1]],
                                     rows2.at[(ch + 1) & 1], gsem)
            pltpu.sync_copy(rows2.at[p], map_sh.at[pixst.at[ch]], add=True)
            pltpu.sync_copy(onesv, cnt_sh.at[pixst.at[ch]], add=True)
        plsc.subcore_barrier()
        out_base = b * HW + sid * MROWS
        pltpu.sync_copy(map_sh.at[pl.ds(sid * MROWS, MROWS)],
                        msum_ref.at[pl.ds(out_base, MROWS)])
        pltpu.sync_copy(cnt_sh.at[pl.ds(sid * MROWS, MROWS)],
                        cnt_ref.at[pl.ds(out_base, MROWS)])


def _phase3_body(y_ref, pixabs_ref, tokloc_ref, w_ref, zrow_ref, zcnt_ref,
                 out_ref,
                 acc_sh, den_sh, pixst, tokst, wb, rows2, wrows2, fbuf, dbuf,
                 gsem, msem, csem):
    cid = lax.axis_index("c")
    sid = lax.axis_index("s")
    for bi in range(BPC):
        b = cid * BPC + bi
        pltpu.sync_copy(zrow_ref.at[pl.ds(0, TROWS)],
                        acc_sh.at[pl.ds(sid * TROWS, TROWS)])
        pltpu.sync_copy(zcnt_ref.at[pl.ds(0, TROWS)],
                        den_sh.at[pl.ds(sid * TROWS, TROWS)])
        row0 = (b * NS + sid) * NCH3
        pltpu.sync_copy(pixabs_ref.at[pl.ds(row0, NCH3)], pixst)
        pltpu.sync_copy(tokloc_ref.at[pl.ds(row0, NCH3)], tokst)
        pltpu.sync_copy(w_ref.at[pl.ds(b * N0 + sid * PPT, PPT)], wb)
        plsc.subcore_barrier()

        g = pltpu.async_copy(y_ref.at[pixst.at[0]], rows2.at[0], gsem)
        for ch in range(NCH3):
            p = ch & 1
            g.wait()
            if ch + 1 < NCH3:
                g = pltpu.async_copy(y_ref.at[pixst.at[ch + 1]],
                                     rows2.at[(ch + 1) & 1], gsem)

            def scale_body(j, _, p=p, ch=ch):
                wj = plsc.load_gather(
                    wb, [jnp.full((L,), ch * CH3, jnp.int32) + j])
                wrows2[p, j, :] = wj
                for c in range(CG):
                    s = pl.ds(c * L, L)
                    rows2[p, j, s] = rows2[p, j, s] * wj
                return 0

            lax.fori_loop(0, CH3, scale_body, 0)
            pltpu.sync_copy(rows2.at[p], acc_sh.at[tokst.at[ch]], add=True)
            pltpu.sync_copy(wrows2.at[p], den_sh.at[tokst.at[ch]], add=True)
        plsc.subcore_barrier()
        tb = sid * TROWS
        pltpu.sync_copy(acc_sh.at[pl.ds(tb, TROWS)], fbuf)
        pltpu.sync_copy(den_sh.at[pl.ds(tb, TROWS)], dbuf)

        def fin_body(j, _):
            r = 1.0 / (dbuf[j, :] + 1e-6)
            for c in range(CG):
                s = pl.ds(c * L, L)
                fbuf[j, s] = fbuf[j, s] * r
            return 0

        lax.fori_loop(0, TROWS, fin_body, 0)
        pltpu.sync_copy(fbuf, out_ref.at[pl.ds(b * N + tb, TROWS)])


def _conv_body(ms_ref, ct_ref, wk_ref, bias_ref, out_ref, pad_ref):
    xm = ms_ref[0] / (ct_ref[0][:, :, 0:1] + 1e-6)
    zr = jnp.zeros((1, W + 2, C), jnp.float32)
    zc = jnp.zeros((H, 1, C), jnp.float32)
    pad_ref[0:1, :, :] = zr
    pad_ref[H + 1:H + 2, :, :] = zr
    pad_ref[1:H + 1, 0:1, :] = zc
    pad_ref[1:H + 1, W + 1:W + 2, :] = zc
    pad_ref[1:H + 1, 1:W + 1, :] = xm
    acc = jnp.broadcast_to(bias_ref[0], (H, W, C))
    for dh in range(3):
        for dw in range(3):
            acc = acc + pad_ref[dh:dh + H, dw:dw + W, :] * wk_ref[dh, dw, :]
    out_ref[0] = acc


def _sc_mesh():
    return plsc.VectorSubcoreMesh(core_axis_name="c", subcore_axis_name="s",
                                  num_cores=NC, num_subcores=NS)


_SC_PARAMS = pltpu.CompilerParams(use_tc_tiling_on_sc=False,
                                  needs_layout_passes=False)


@jax.jit
def _run(x2, locx, locy, tokf, wf, wk, bias):
    zrow = jnp.zeros((MROWS, C), _f32)
    zcnt = jnp.zeros((MROWS, CW), _f32)
    ones_rows = jnp.ones((CH1, CW), _f32)

    pixloc, tokabs, pixabs = pl.pallas_call(
        _idx_body,
        out_shape=(jax.ShapeDtypeStruct((B, N0), jnp.int32),
                   jax.ShapeDtypeStruct((B, N0), jnp.int32),
                   jax.ShapeDtypeStruct((B, N0), jnp.int32)),
    )(locx.reshape(B, N0), locy.reshape(B, N0), tokf.reshape(B, N0))

    phase1 = pl.kernel(
        _phase1_body,
        out_type=(jax.ShapeDtypeStruct((B * HW, C), _f32),
                  jax.ShapeDtypeStruct((B * HW, CW), _f32)),
        mesh=_sc_mesh(),
        compiler_params=_SC_PARAMS,
        scratch_types=[
            pltpu.VMEM_SHARED((HW, C), _f32),
            pltpu.VMEM_SHARED((HW, CW), _f32),
            pltpu.VMEM((NCH1, CH1), jnp.int32),
            pltpu.VMEM((NCH1, CH1), jnp.int32),
            pltpu.VMEM((2, CH1, C), _f32),
            pltpu.VMEM((CH1, CW), _f32),
            pltpu.SemaphoreType.DMA,
            pltpu.SemaphoreType.DMA,
            pltpu.SemaphoreType.DMA,
        ],
    )
    msum, cnt = phase1(x2, pixloc.reshape(B * NS * NCH1, CH1),
                       tokabs.reshape(B * NS * NCH1, CH1),
                       zrow, zcnt, ones_rows)

    y = pl.pallas_call(
        _conv_body,
        grid=(B,),
        in_specs=[
            pl.BlockSpec((1, H, W, C), lambda b: (b, 0, 0, 0)),
            pl.BlockSpec((1, H, W, CW), lambda b: (b, 0, 0, 0)),
            pl.BlockSpec((3, 3, C), lambda b: (0, 0, 0)),
            pl.BlockSpec((1, C), lambda b: (0, 0)),
        ],
        out_specs=pl.BlockSpec((1, H, W, C), lambda b: (b, 0, 0, 0)),
        out_shape=jax.ShapeDtypeStruct((B, H, W, C), _f32),
        scratch_shapes=[pltpu.VMEM((H + 2, W + 2, C), _f32)],
    )(msum.reshape(B, H, W, C), cnt.reshape(B, H, W, CW), wk,
      bias.reshape(1, C))

    phase3 = pl.kernel(
        _phase3_body,
        out_type=jax.ShapeDtypeStruct((B * N, C), _f32),
        mesh=_sc_mesh(),
        compiler_params=_SC_PARAMS,
        scratch_types=[
            pltpu.VMEM_SHARED((N, C), _f32),
            pltpu.VMEM_SHARED((N, CW), _f32),
            pltpu.VMEM((NCH3, CH3), jnp.int32),
            pltpu.VMEM((NCH3, CH3), jnp.int32),
            pltpu.VMEM((PPT,), _f32),
            pltpu.VMEM((2, CH3, C), _f32),
            pltpu.VMEM((2, CH3, CW), _f32),
            pltpu.VMEM((TROWS, C), _f32),
            pltpu.VMEM((TROWS, CW), _f32),
            pltpu.SemaphoreType.DMA,
            pltpu.SemaphoreType.DMA,
            pltpu.SemaphoreType.DMA,
        ],
    )
    out = phase3(y.reshape(B * HW, C), pixabs.reshape(B * NS * NCH3, CH3),
                 tokf.reshape(B * NS * NCH3, CH3), wf, zrow, zcnt)
    return out.reshape(B, N, C)


def kernel(x, loc, loc_orig, idx_agg, agg_weight, H_, W_, dw_weight, dw_bias):
    del loc
    x2 = x.reshape(B * N, C)
    locx = loc_orig[..., 0].reshape(B * N0)
    locy = loc_orig[..., 1].reshape(B * N0)
    tokf = idx_agg.astype(jnp.int32).reshape(B * N0)
    wf = agg_weight.astype(_f32).reshape(B * N0)
    wk = jnp.transpose(dw_weight[:, 0], (1, 2, 0))
    return _run(x2, locx, locy, tokf, wf, wk, dw_bias.astype(_f32))


# speedup vs baseline: 7.4038x; 1.5286x over previous
import jax
import jax.numpy as jnp
from jax import lax
from jax.experimental import pallas as pl
from jax.experimental.pallas import tpu as pltpu
from jax.experimental.pallas import tpu_sc as plsc

B, N, C, N0 = 8, 4096, 96, 16384
H, W = 128, 128
HW = H * W

NC, NS, L = 2, 16, 16
BPC = B // NC
PPT = N0 // NS
CH1 = 64
NCH1 = PPT // CH1
CH3 = 128
NCH3 = PPT // CH3
MROWS = HW // NS
TROWS = N // NS
CW = 16
CG = C // L

_f32 = jnp.float32


def _idx_body(lx_ref, ly_ref, tok_ref, pixloc_ref, tokabs_ref, pixabs_ref):
    def pix1d(v, d):
        v = jnp.minimum(jnp.maximum(v, -1.0), 1.0)
        t = 0.5 * (v + 1.0) * float(d) - 0.5
        i = jnp.round(t).astype(jnp.int32)
        return jnp.minimum(jnp.maximum(i, 0), d - 1)

    xi = pix1d(lx_ref[...], W)
    yi = pix1d(ly_ref[...], H)
    pix = xi + yi * W
    boff = lax.broadcasted_iota(jnp.int32, (B, N0), 0)
    pixloc_ref[...] = pix
    tokabs_ref[...] = tok_ref[...] + boff * N
    pixabs_ref[...] = pix + boff * HW


def _phase1_body(x_ref, pixloc_ref, tokabs_ref, zrow_ref, zcnt_ref, ones_ref,
                 mc_ref,
                 map_sh, cnt_sh, pixst, tokst, rows2, onesv,
                 gsem, fsem, csem):
    cid = lax.axis_index("c")
    sid = lax.axis_index("s")
    pltpu.sync_copy(ones_ref, onesv)
    fl = []
    for bi in range(BPC):
        b = cid * BPC + bi
        row0 = (b * NS + sid) * NCH1
        pltpu.sync_copy(pixloc_ref.at[pl.ds(row0, NCH1)], pixst)
        pltpu.sync_copy(tokabs_ref.at[pl.ds(row0, NCH1)], tokst)
        for d in fl:
            d.wait()
        z0 = pltpu.async_copy(zrow_ref, map_sh.at[pl.ds(sid * MROWS, MROWS)],
                              fsem)
        z1 = pltpu.async_copy(zcnt_ref, cnt_sh.at[pl.ds(sid * MROWS, MROWS)],
                              csem)
        z0.wait()
        z1.wait()
        plsc.subcore_barrier()

        g = pltpu.async_copy(x_ref.at[tokst.at[0]], rows2.at[0], gsem)
        cds = []
        for ch in range(NCH1):
            p = ch & 1
            g.wait()
            if ch + 1 < NCH1:
                g = pltpu.async_copy(x_ref.at[tokst.at[ch + 1]],
                                     rows2.at[(ch + 1) & 1], gsem)
            pltpu.sync_copy(rows2.at[p], map_sh.at[pixst.at[ch]], add=True)
            cds.append(pltpu.async_copy(onesv, cnt_sh.at[pixst.at[ch]], csem,
                                        add=True))
        for d in cds:
            d.wait()
        plsc.subcore_barrier()
        out_base = b * HW + sid * MROWS
        fl = [pltpu.async_copy(map_sh.at[pl.ds(sid * MROWS, MROWS)],
                               mc_ref.at[pl.ds(out_base, MROWS), pl.ds(0, C)],
                               fsem),
              pltpu.async_copy(cnt_sh.at[pl.ds(sid * MROWS, MROWS)],
                               mc_ref.at[pl.ds(out_base, MROWS),
                                         pl.ds(C, CW)], csem)]
    for d in fl:
        d.wait()


def _phase3_body(y_ref, pixabs_ref, tokloc_ref, w_ref, zrow_ref, zcnt_ref,
                 out_ref,
                 acc_sh, den_sh, pixst, tokst, wb, rows2, wrows2, fbuf, dbuf,
                 gsem, msem, csem, osem):
    cid = lax.axis_index("c")
    sid = lax.axis_index("s")
    ofl = None
    for bi in range(BPC):
        b = cid * BPC + bi
        z0 = pltpu.async_copy(zrow_ref, acc_sh.at[pl.ds(sid * TROWS, TROWS)],
                              msem)
        z1 = pltpu.async_copy(zcnt_ref.at[pl.ds(0, TROWS)],
                              den_sh.at[pl.ds(sid * TROWS, TROWS)], csem)
        row0 = (b * NS + sid) * NCH3
        pltpu.sync_copy(pixabs_ref.at[pl.ds(row0, NCH3)], pixst)
        pltpu.sync_copy(tokloc_ref.at[pl.ds(row0, NCH3)], tokst)
        pltpu.sync_copy(w_ref.at[pl.ds(b * N0 + sid * PPT, PPT)], wb)
        z0.wait()
        z1.wait()
        plsc.subcore_barrier()

        g = pltpu.async_copy(y_ref.at[pixst.at[0]], rows2.at[0], gsem)
        for ch in range(NCH3):
            p = ch & 1
            g.wait()
            if ch + 1 < NCH3:
                g = pltpu.async_copy(y_ref.at[pixst.at[ch + 1]],
                                     rows2.at[(ch + 1) & 1], gsem)

            def scale_body(j, _, p=p, ch=ch):
                wj = plsc.load_gather(
                    wb, [jnp.full((L,), ch * CH3, jnp.int32) + j])
                wrows2[p, j, :] = wj
                for c in range(CG):
                    s = pl.ds(c * L, L)
                    rows2[p, j, s] = rows2[p, j, s] * wj
                return 0

            lax.fori_loop(0, CH3, scale_body, 0)
            pltpu.sync_copy(rows2.at[p], acc_sh.at[tokst.at[ch]], add=True)
            pltpu.sync_copy(wrows2.at[p], den_sh.at[tokst.at[ch]], add=True)
        plsc.subcore_barrier()
        tb = sid * TROWS
        if ofl is not None:
            ofl.wait()
        pltpu.sync_copy(acc_sh.at[pl.ds(tb, TROWS)], fbuf)
        pltpu.sync_copy(den_sh.at[pl.ds(tb, TROWS)], dbuf)

        def fin_body(j, _):
            r = 1.0 / (dbuf[j, :] + 1e-6)
            for c in range(CG):
                s = pl.ds(c * L, L)
                fbuf[j, s] = fbuf[j, s] * r
            return 0

        lax.fori_loop(0, TROWS, fin_body, 0)
        ofl = pltpu.async_copy(fbuf, out_ref.at[pl.ds(b * N + tb, TROWS)],
                               osem)
    ofl.wait()


def _conv_body(mc_ref, wk_ref, bias_ref, out_ref, pad_ref):
    mc = mc_ref[...]
    xm = (mc[:, 0:C] / (mc[:, C:C + 1] + 1e-6)).reshape(H, W, C)
    zr = jnp.zeros((1, W + 2, C), jnp.float32)
    zc = jnp.zeros((H, 1, C), jnp.float32)
    pad_ref[0:1, :, :] = zr
    pad_ref[H + 1:H + 2, :, :] = zr
    pad_ref[1:H + 1, 0:1, :] = zc
    pad_ref[1:H + 1, W + 1:W + 2, :] = zc
    pad_ref[1:H + 1, 1:W + 1, :] = xm
    acc = jnp.broadcast_to(bias_ref[0], (H, W, C))
    for dh in range(3):
        for dw in range(3):
            acc = acc + pad_ref[dh:dh + H, dw:dw + W, :] * wk_ref[dh, dw, :]
    out_ref[:, 0:C] = acc.reshape(HW, C)
    out_ref[:, C:128] = jnp.zeros((HW, 128 - C), jnp.float32)


def _sc_mesh():
    return plsc.VectorSubcoreMesh(core_axis_name="c", subcore_axis_name="s",
                                  num_cores=NC, num_subcores=NS)


_SC_PARAMS = pltpu.CompilerParams(use_tc_tiling_on_sc=False,
                                  needs_layout_passes=False)


@jax.jit
def _run(x2, locx, locy, tokf, wf, wk, bias):
    zrow = jnp.zeros((MROWS, C), _f32)
    zcnt = jnp.zeros((MROWS, CW), _f32)
    ones_rows = jnp.ones((CH1, CW), _f32)

    pixloc, tokabs, pixabs = pl.pallas_call(
        _idx_body,
        out_shape=(jax.ShapeDtypeStruct((B, N0), jnp.int32),
                   jax.ShapeDtypeStruct((B, N0), jnp.int32),
                   jax.ShapeDtypeStruct((B, N0), jnp.int32)),
    )(locx.reshape(B, N0), locy.reshape(B, N0), tokf.reshape(B, N0))

    phase1 = pl.kernel(
        _phase1_body,
        out_type=jax.ShapeDtypeStruct((B * HW, 128), _f32),
        mesh=_sc_mesh(),
        compiler_params=_SC_PARAMS,
        scratch_types=[
            pltpu.VMEM_SHARED((HW, C), _f32),
            pltpu.VMEM_SHARED((HW, CW), _f32),
            pltpu.VMEM((NCH1, CH1), jnp.int32),
            pltpu.VMEM((NCH1, CH1), jnp.int32),
            pltpu.VMEM((2, CH1, C), _f32),
            pltpu.VMEM((CH1, CW), _f32),
            pltpu.SemaphoreType.DMA,
            pltpu.SemaphoreType.DMA,
            pltpu.SemaphoreType.DMA,
        ],
    )
    mc = phase1(x2, pixloc.reshape(B * NS * NCH1, CH1),
                tokabs.reshape(B * NS * NCH1, CH1),
                zrow, zcnt, ones_rows)

    y = pl.pallas_call(
        _conv_body,
        grid=(B,),
        in_specs=[
            pl.BlockSpec((HW, 128), lambda b: (b, 0)),
            pl.BlockSpec((3, 3, C), lambda b: (0, 0, 0)),
            pl.BlockSpec((1, C), lambda b: (0, 0)),
        ],
        out_specs=pl.BlockSpec((HW, 128), lambda b: (b, 0)),
        out_shape=jax.ShapeDtypeStruct((B * HW, 128), _f32),
        scratch_shapes=[pltpu.VMEM((H + 2, W + 2, C), _f32)],
    )(mc, wk, bias.reshape(1, C))

    phase3 = pl.kernel(
        _phase3_body,
        out_type=jax.ShapeDtypeStruct((B * N, 128), _f32),
        mesh=_sc_mesh(),
        compiler_params=_SC_PARAMS,
        scratch_types=[
            pltpu.VMEM_SHARED((N, 128), _f32),
            pltpu.VMEM_SHARED((N, CW), _f32),
            pltpu.VMEM((NCH3, CH3), jnp.int32),
            pltpu.VMEM((NCH3, CH3), jnp.int32),
            pltpu.VMEM((PPT,), _f32),
            pltpu.VMEM((2, CH3, 128), _f32),
            pltpu.VMEM((2, CH3, CW), _f32),
            pltpu.VMEM((TROWS, 128), _f32),
            pltpu.VMEM((TROWS, CW), _f32),
            pltpu.SemaphoreType.DMA,
            pltpu.SemaphoreType.DMA,
            pltpu.SemaphoreType.DMA,
            pltpu.SemaphoreType.DMA,
        ],
    )
    zacc = jnp.zeros((TROWS, 128), _f32)
    out = phase3(y, pixabs.reshape(B * NS * NCH3, CH3),
                 tokf.reshape(B * NS * NCH3, CH3), wf, zacc, zcnt)
    return out[:, 0:C].reshape(B, N, C)


def kernel(x, loc, loc_orig, idx_agg, agg_weight, H_, W_, dw_weight, dw_bias):
    del loc
    x2 = x.reshape(B * N, C)
    locx = loc_orig[..., 0].reshape(B * N0)
    locy = loc_orig[..., 1].reshape(B * N0)
    tokf = idx_agg.astype(jnp.int32).reshape(B * N0)
    wf = agg_weight.astype(_f32).reshape(B * N0)
    wk = jnp.transpose(dw_weight[:, 0], (1, 2, 0))
    return _run(x2, locx, locy, tokf, wf, wk, dw_bias.astype(_f32))
